# baseline (speedup 1.0000x reference)
import functools

import jax
import jax.numpy as jnp
from jax import lax
from jax.experimental import pallas as pl
from jax.experimental.pallas import tpu as pltpu


def _conv_stats_kernel(x_ref, wc_ref, y_ref, stats_ref, p_ref, *,
                       H, W, Cin, Cout, IMG):
    HW = H * W
    lane = lax.broadcasted_iota(jnp.int32, (1, HW), 1)
    col = lane % W
    not_last = col != (W - 1)
    not_first = col != 0
    row_lo = lane >= W
    row_hi = lane < (HW - W)

    s_acc = jnp.zeros((Cout, 1), jnp.float32)
    ss_acc = jnp.zeros((Cout, 1), jnp.float32)
    for i in range(IMG):
        p = p_ref.at[i % 2]
        xb = x_ref[i]
        xm = jnp.where(not_last, xb, 0).astype(jnp.float32)
        xp = jnp.where(not_first, xb, 0).astype(jnp.float32)
        p[0:Cin, :] = pltpu.roll(xm, 1, axis=1).astype(jnp.bfloat16)
        p[Cin:2 * Cin, :] = xb
        p[2 * Cin:3 * Cin, :] = pltpu.roll(xp, HW - 1, axis=1).astype(
            jnp.bfloat16)

        u = jnp.dot(wc_ref[...], p_ref[i % 2],
                    preferred_element_type=jnp.float32)
        y = u[Cout:2 * Cout]
        y = y + jnp.where(row_lo, pltpu.roll(u[0:Cout], W, axis=1), 0.0)
        y = y + jnp.where(row_hi,
                          pltpu.roll(u[2 * Cout:3 * Cout], HW - W, axis=1),
                          0.0)
        s_acc += jnp.sum(y, axis=1, keepdims=True)
        ss_acc += jnp.sum(y * y, axis=1, keepdims=True)
        y_ref[i] = y.astype(jnp.bfloat16)
    stats_ref[...] = jnp.concatenate([s_acc, ss_acc], axis=1)


def _bn_relu_kernel(y_ref, sc_ref, sh_ref, o_ref, *, IMG):
    for i in range(IMG):
        y = y_ref[i].astype(jnp.float32)
        o_ref[i] = jnp.maximum(y * sc_ref[...] + sh_ref[...],
                               0.0).astype(jnp.bfloat16)


def kernel(x, weight, bias, gamma, beta, *, eps=1e-5):
    N, Cin, H, W = x.shape
    Cout = weight.shape[0]
    HW = H * W
    IMG = 4 if N % 4 == 0 else (2 if N % 2 == 0 else 1)
    NB = N // IMG

    xq = x.reshape(N, Cin, HW).astype(jnp.bfloat16)
    wc = jnp.transpose(weight, (2, 0, 3, 1)).reshape(3 * Cout, 3 * Cin)
    wc = wc.astype(jnp.bfloat16)

    vmem_limit = 56 * 1024 * 1024

    y, stats = pl.pallas_call(
        functools.partial(_conv_stats_kernel, H=H, W=W, Cin=Cin, Cout=Cout,
                          IMG=IMG),
        grid=(NB,),
        in_specs=[
            pl.BlockSpec((IMG, Cin, HW), lambda n: (n, 0, 0)),
            pl.BlockSpec((3 * Cout, 3 * Cin), lambda n: (0, 0)),
        ],
        out_specs=(
            pl.BlockSpec((IMG, Cout, HW), lambda n: (n, 0, 0)),
            pl.BlockSpec((None, Cout, 2), lambda n: (n, 0, 0)),
        ),
        out_shape=(
            jax.ShapeDtypeStruct((N, Cout, HW), jnp.bfloat16),
            jax.ShapeDtypeStruct((NB, Cout, 2), jnp.float32),
        ),
        scratch_shapes=[pltpu.VMEM((2, 3 * Cin, HW), jnp.bfloat16)],
        compiler_params=pltpu.CompilerParams(
            dimension_semantics=("parallel",),
            vmem_limit_bytes=vmem_limit),
    )(xq, wc)

    count = jnp.float32(N * H * W)
    tot = jnp.sum(stats, axis=0)
    mean = tot[:, 0] / count
    var = jnp.maximum(tot[:, 1] / count - mean * mean, 0.0)
    inv = lax.rsqrt(var + eps)
    scale = (gamma * inv).reshape(Cout, 1)
    shift = (beta - mean * gamma * inv).reshape(Cout, 1)

    outq = pl.pallas_call(
        functools.partial(_bn_relu_kernel, IMG=IMG),
        grid=(NB,),
        in_specs=[
            pl.BlockSpec((IMG, Cout, HW), lambda n: (n, 0, 0)),
            pl.BlockSpec((Cout, 1), lambda n: (0, 0)),
            pl.BlockSpec((Cout, 1), lambda n: (0, 0)),
        ],
        out_specs=pl.BlockSpec((IMG, Cout, HW), lambda n: (n, 0, 0)),
        out_shape=jax.ShapeDtypeStruct((N, Cout, HW), jnp.bfloat16),
        compiler_params=pltpu.CompilerParams(
            dimension_semantics=("parallel",),
            vmem_limit_bytes=vmem_limit),
    )(y, scale, shift)

    return outq.astype(jnp.float32).reshape(N, Cout, H, W)


# speedup vs baseline: 1.1080x; 1.1080x over previous
import functools

import jax
import jax.numpy as jnp
from jax import lax
from jax.experimental import pallas as pl
from jax.experimental.pallas import tpu as pltpu


def _conv_stats_kernel(x_ref, wc_ref, y_ref, stats_ref, p_ref, *,
                       H, W, Cin, Cout, IMG):
    HW = H * W
    lane = lax.broadcasted_iota(jnp.int32, (1, HW), 1)
    col = lane % W
    not_last = col != (W - 1)
    not_first = col != 0
    row_lo = lane >= W
    row_hi = lane < (HW - W)

    s_acc = jnp.zeros((Cout, 1), jnp.float32)
    ss_acc = jnp.zeros((Cout, 1), jnp.float32)
    for i in range(IMG):
        p = p_ref.at[i % 2]
        xb = x_ref[i]
        xm = jnp.where(not_last, xb, 0.0)
        xp = jnp.where(not_first, xb, 0.0)
        p[0:Cin, :] = pltpu.roll(xm, 1, axis=1).astype(jnp.bfloat16)
        p[Cin:2 * Cin, :] = xb.astype(jnp.bfloat16)
        p[2 * Cin:3 * Cin, :] = pltpu.roll(xp, HW - 1, axis=1).astype(
            jnp.bfloat16)

        u = jnp.dot(wc_ref[...], p_ref[i % 2],
                    preferred_element_type=jnp.float32)
        y = u[Cout:2 * Cout]
        y = y + jnp.where(row_lo, pltpu.roll(u[0:Cout], W, axis=1), 0.0)
        y = y + jnp.where(row_hi,
                          pltpu.roll(u[2 * Cout:3 * Cout], HW - W, axis=1),
                          0.0)
        s_acc += jnp.sum(y, axis=1, keepdims=True)
        ss_acc += jnp.sum(y * y, axis=1, keepdims=True)
        y_ref[i] = y.astype(jnp.bfloat16)
    stats_ref[...] = jnp.concatenate([s_acc, ss_acc], axis=1)


def _bn_relu_kernel(y_ref, sc_ref, sh_ref, o_ref, *, IMG):
    for i in range(IMG):
        y = y_ref[i].astype(jnp.float32)
        o_ref[i] = jnp.maximum(y * sc_ref[...] + sh_ref[...],
                               0.0).astype(jnp.bfloat16)


def kernel(x, weight, bias, gamma, beta, *, eps=1e-5):
    N, Cin, H, W = x.shape
    Cout = weight.shape[0]
    HW = H * W
    IMG = 4 if N % 4 == 0 else (2 if N % 2 == 0 else 1)
    NB = N // IMG

    xq = x.reshape(N, Cin, HW)
    wc = jnp.transpose(weight, (2, 0, 3, 1)).reshape(3 * Cout, 3 * Cin)
    wc = wc.astype(jnp.bfloat16)

    vmem_limit = 56 * 1024 * 1024

    y, stats = pl.pallas_call(
        functools.partial(_conv_stats_kernel, H=H, W=W, Cin=Cin, Cout=Cout,
                          IMG=IMG),
        grid=(NB,),
        in_specs=[
            pl.BlockSpec((IMG, Cin, HW), lambda n: (n, 0, 0)),
            pl.BlockSpec((3 * Cout, 3 * Cin), lambda n: (0, 0)),
        ],
        out_specs=(
            pl.BlockSpec((IMG, Cout, HW), lambda n: (n, 0, 0)),
            pl.BlockSpec((None, Cout, 2), lambda n: (n, 0, 0)),
        ),
        out_shape=(
            jax.ShapeDtypeStruct((N, Cout, HW), jnp.bfloat16),
            jax.ShapeDtypeStruct((NB, Cout, 2), jnp.float32),
        ),
        scratch_shapes=[pltpu.VMEM((2, 3 * Cin, HW), jnp.bfloat16)],
        compiler_params=pltpu.CompilerParams(
            dimension_semantics=("parallel",),
            vmem_limit_bytes=vmem_limit),
    )(xq, wc)

    count = jnp.float32(N * H * W)
    tot = jnp.sum(stats, axis=0)
    mean = tot[:, 0] / count
    var = jnp.maximum(tot[:, 1] / count - mean * mean, 0.0)
    inv = lax.rsqrt(var + eps)
    scale = (gamma * inv).reshape(Cout, 1)
    shift = (beta - mean * gamma * inv).reshape(Cout, 1)

    outq = pl.pallas_call(
        functools.partial(_bn_relu_kernel, IMG=IMG),
        grid=(NB,),
        in_specs=[
            pl.BlockSpec((IMG, Cout, HW), lambda n: (n, 0, 0)),
            pl.BlockSpec((Cout, 1), lambda n: (0, 0)),
            pl.BlockSpec((Cout, 1), lambda n: (0, 0)),
        ],
        out_specs=pl.BlockSpec((IMG, Cout, HW), lambda n: (n, 0, 0)),
        out_shape=jax.ShapeDtypeStruct((N, Cout, HW), jnp.bfloat16),
        compiler_params=pltpu.CompilerParams(
            dimension_semantics=("parallel",),
            vmem_limit_bytes=vmem_limit),
    )(y, scale, shift)

    return outq.astype(jnp.float32).reshape(N, Cout, H, W)
